# baseline (speedup 1.0000x reference)
import jax
import jax.numpy as jnp
import numpy as np
from jax import lax
from jax.experimental import pallas as pl
from jax.experimental.pallas import tpu as pltpu
from jax.experimental.pallas import tpu_sc as plsc

B = 32
N = 1000000

BLK = 2048
CH = 128
NCH = BLK // CH

W_SC = 22528
NSLICE = 8
L_SC = W_SC * NSLICE
C0 = 400 * BLK
SC_END = C0 + L_SC

NFULL = C0 // BLK
TBLK = SC_END // BLK
NB = NFULL + 1
TAIL = N - SC_END
NCH_TAIL = (TAIL + CH - 1) // CH

SC_NC = 2
SC_NS = 16
NW = SC_NC * SC_NS
LANES = 16
RG = 8
CSC = 2048
NCHUNK = W_SC // CSC
GROUPS = CSC // LANES
NSET = 2
GPAIRS = GROUPS // NSET
SC_OUT = B * NSLICE * NSET * LANES

_TINY = np.float32(np.finfo(np.float32).tiny)
_K0 = np.uint32(0)
_K1 = np.uint32(42)
_K2 = np.uint32(np.uint32(0) ^ np.uint32(42) ^ np.uint32(0x1BD11BDA))
_KS = (_K0, _K1, _K2)
_ROT = ((13, 15, 26, 6), (17, 29, 16, 24))
_INJ = ((1, 2), (2, 0), (0, 1), (1, 2), (2, 0))


def _rotl(x, r):
    return (x << np.uint32(r)) | (x >> np.uint32(32 - r))


def _threefry_bits(cnt):
    x1i = cnt + _KS[1]
    x0 = x1i
    x1 = _rotl(x1i, _ROT[0][0]) ^ x1i
    first = True
    for g in range(5):
        for r in _ROT[g % 2]:
            if first:
                first = False
                continue
            x0 = x0 + x1
            x1 = _rotl(x1, r) ^ x0
        a, b = _INJ[g]
        x0 = x0 + _KS[a]
        x1 = x1 + (_KS[b] + np.uint32(g + 1))
    return x0 ^ x1


def _gumbel_from_bits(bits):
    fb = (bits >> np.uint32(9)) | np.uint32(0x3F800000)
    fl = lax.bitcast_convert_type(fb, jnp.float32) - jnp.float32(1.0)
    u = jnp.maximum(fl, _TINY)
    return -jnp.log(-jnp.log(u))


def _chunk_update(x, col, masked,
                  pmax_ref, pidx_ref, m_ref, s_ref, sl):
    flat = (lax.broadcasted_iota(jnp.uint32, (B, CH), 0) * np.uint32(N)
            + col.astype(jnp.uint32))
    g = _gumbel_from_bits(_threefry_bits(flat))
    p = x + g
    if masked:
        valid = col < N
        p = jnp.where(valid, p, -jnp.inf)
        xs = jnp.where(valid, x, -jnp.inf)
    else:
        xs = x

    pm = pmax_ref[:, sl]
    upd = p > pm
    pmax_ref[:, sl] = jnp.where(upd, p, pm)
    pidx_ref[:, sl] = jnp.where(upd, col, pidx_ref[:, sl])

    mo = m_ref[:, sl]
    nm = jnp.maximum(mo, xs)
    s_ref[:, sl] = s_ref[:, sl] * jnp.exp(mo - nm) + jnp.exp(xs - nm)
    m_ref[:, sl] = nm


def _tc_body(x_ref, pmax_o, pidx_o, pick_o, m_o, s_o,
             pmax_ref, pidx_ref, m_ref, s_ref):
    c = pl.program_id(0)

    @pl.when(c == 0)
    def _init():
        pmax_ref[...] = jnp.full((B, BLK), -jnp.inf, jnp.float32)
        pidx_ref[...] = jnp.zeros((B, BLK), jnp.int32)
        m_ref[...] = jnp.full((B, BLK), -jnp.inf, jnp.float32)
        s_ref[...] = jnp.zeros((B, BLK), jnp.float32)

    ch_iota = lax.broadcasted_iota(jnp.int32, (B, CH), 1)

    @pl.when(c < NFULL)
    def _full():
        base = c * BLK
        for k in range(NCH):
            sl = slice(k * CH, (k + 1) * CH)
            col = (base + k * CH) + ch_iota
            _chunk_update(x_ref[:, sl], col, False,
                          pmax_ref, pidx_ref, m_ref, s_ref, sl)

    @pl.when(c == NFULL)
    def _tail():
        base = TBLK * BLK
        for k in range(NCH_TAIL):
            sl = slice(k * CH, (k + 1) * CH)
            col = (base + k * CH) + ch_iota
            _chunk_update(x_ref[:, sl], col, True,
                          pmax_ref, pidx_ref, m_ref, s_ref, sl)

        pmax_v = pmax_ref[...]
        pidx_v = pidx_ref[...]
        m_v = m_ref[...]
        s_v = s_ref[...]

        bpm = jnp.max(pmax_v, axis=1)
        winners = pmax_v == bpm[:, None]
        idx = jnp.min(jnp.where(winners, pidx_v, jnp.int32(2**30)), axis=1)

        flatw = (lax.broadcasted_iota(jnp.uint32, (B,), 0) * np.uint32(N)
                 + idx.astype(jnp.uint32))
        pick = bpm - _gumbel_from_bits(_threefry_bits(flatw))

        mf = jnp.max(m_v, axis=1)
        z = jnp.sum(s_v * jnp.exp(m_v - mf[:, None]), axis=1)

        pmax_o[...] = bpm
        pidx_o[...] = idx
        pick_o[...] = pick
        m_o[...] = mf
        s_o[...] = z


def _tc_partials(logits):
    return pl.pallas_call(
        _tc_body,
        grid=(NB,),
        in_specs=[pl.BlockSpec(
            (B, BLK), lambda c: (0, jnp.where(c == NFULL, TBLK, c)))],
        out_specs=[pl.BlockSpec((B,), lambda c: (0,))] * 5,
        out_shape=[
            jax.ShapeDtypeStruct((B,), jnp.float32),
            jax.ShapeDtypeStruct((B,), jnp.int32),
            jax.ShapeDtypeStruct((B,), jnp.float32),
            jax.ShapeDtypeStruct((B,), jnp.float32),
            jax.ShapeDtypeStruct((B,), jnp.float32),
        ],
        scratch_shapes=[
            pltpu.VMEM((B, BLK), jnp.float32),
            pltpu.VMEM((B, BLK), jnp.int32),
            pltpu.VMEM((B, BLK), jnp.float32),
            pltpu.VMEM((B, BLK), jnp.float32),
        ],
    )(logits)


_LOG_P = (0.09152395278215408, -0.13937802612781525, 0.14647668600082397,
          -0.16609898209571838, 0.19987723231315613, -0.25000593066215515,
          0.3333338797092438)
_SQRTHF = np.float32(0.70710678118654752440)


def _sc_logf(x):
    bits = lax.bitcast_convert_type(x, jnp.uint32)
    e = (bits >> np.uint32(23)).astype(jnp.int32) - 127
    mb = (bits & np.uint32(0x7FFFFF)) | np.uint32(0x3F800000)
    m = lax.bitcast_convert_type(mb, jnp.float32)
    m01 = m * jnp.float32(0.5)
    small = m01 < _SQRTHF
    e2 = jnp.where(small, e, e + 1)
    f = jnp.where(small, m - jnp.float32(1.0), m01 - jnp.float32(1.0))
    z = f * f
    y = jnp.full_like(f, _LOG_P[0])
    for cc in _LOG_P[1:]:
        y = y * f + jnp.float32(cc)
    y = y * f * z
    ef = e2.astype(jnp.float32)
    y = y + ef * jnp.float32(-2.12194440e-4)
    y = y - jnp.float32(0.5) * z
    return (f + y) + ef * jnp.float32(0.693359375)


def _sc_gumbel_from_bits(bits):
    fb = (bits >> np.uint32(9)) | np.uint32(0x3F800000)
    fl = lax.bitcast_convert_type(fb, jnp.float32) - jnp.float32(1.0)
    u = jnp.maximum(fl, _TINY)
    return -_sc_logf(-_sc_logf(u))


def _sc_body(logits_hbm, pmax_hbm, pidx_hbm, m_hbm, s_hbm,
             buf, apm, aix, am, asum, stage_f, stage_i):
    wid = lax.axis_index("s") * SC_NC + lax.axis_index("c")
    a = wid // NSLICE
    bslc = wid % NSLICE
    row0 = a * RG
    colw0 = C0 + bslc * W_SC
    lane = lax.iota(jnp.int32, LANES)

    for r in range(RG):
        for k in range(NSET):
            apm[r, k] = jnp.full((LANES,), -jnp.inf, jnp.float32)
            aix[r, k] = jnp.zeros((LANES,), jnp.int32)
            am[r, k] = jnp.full((LANES,), -jnp.inf, jnp.float32)
            asum[r, k] = jnp.zeros((LANES,), jnp.float32)

    def chunk(i, _):
        colbase = colw0 + i * CSC
        pltpu.sync_copy(
            logits_hbm.at[pl.ds(row0, RG), pl.ds(colbase, CSC)], buf)

        def per_row(r, _):
            rowN = (row0 + r) * N

            def grp(j, carry):
                out = []
                for k in range(NSET):
                    pm, pix, m, s = carry[4 * k:4 * k + 4]
                    jj = j * NSET + k
                    x = buf[r, pl.ds(jj * LANES, LANES)]
                    col = (colbase + jj * LANES) + lane
                    flat = (rowN + col).astype(jnp.uint32)
                    g = _sc_gumbel_from_bits(_threefry_bits(flat))
                    p = x + g
                    upd = p > pm
                    pm = jnp.where(upd, p, pm)
                    pix = jnp.where(upd, col, pix)
                    nm = jnp.maximum(m, x)
                    s = s * jnp.exp(m - nm) + jnp.exp(x - nm)
                    out += [pm, pix, nm, s]
                return tuple(out)

            carry = tuple(ref[r, k]
                          for k in range(NSET)
                          for ref in (apm, aix, am, asum))
            res = lax.fori_loop(0, GPAIRS, grp, carry, unroll=4)
            for k in range(NSET):
                apm[r, k], aix[r, k], am[r, k], asum[r, k] = (
                    res[4 * k:4 * k + 4])
            return 0

        return lax.fori_loop(0, RG, per_row, 0)

    lax.fori_loop(0, NCHUNK, chunk, 0)

    for r in range(RG):
        for k in range(NSET):
            off = (((row0 + r) * NSLICE + bslc) * NSET + k) * LANES
            stage_f[...] = apm[r, k]
            pltpu.sync_copy(stage_f, pmax_hbm.at[pl.ds(off, LANES)])
            stage_i[...] = aix[r, k]
            pltpu.sync_copy(stage_i, pidx_hbm.at[pl.ds(off, LANES)])
            stage_f[...] = am[r, k]
            pltpu.sync_copy(stage_f, m_hbm.at[pl.ds(off, LANES)])
            stage_f[...] = asum[r, k]
            pltpu.sync_copy(stage_f, s_hbm.at[pl.ds(off, LANES)])


def _sc_partials(logits):
    mesh = plsc.VectorSubcoreMesh(core_axis_name="c", subcore_axis_name="s")
    f32 = jnp.float32
    run = pl.kernel(
        _sc_body,
        out_type=[
            jax.ShapeDtypeStruct((SC_OUT,), f32),
            jax.ShapeDtypeStruct((SC_OUT,), jnp.int32),
            jax.ShapeDtypeStruct((SC_OUT,), f32),
            jax.ShapeDtypeStruct((SC_OUT,), f32),
        ],
        mesh=mesh,
        scratch_types=[
            pltpu.VMEM((RG, CSC), f32),
            pltpu.VMEM((RG, NSET, LANES), f32),
            pltpu.VMEM((RG, NSET, LANES), jnp.int32),
            pltpu.VMEM((RG, NSET, LANES), f32),
            pltpu.VMEM((RG, NSET, LANES), f32),
            pltpu.VMEM((LANES,), f32),
            pltpu.VMEM((LANES,), jnp.int32),
        ],
    )
    return run(logits)


SLOTS = NSLICE * NSET * LANES


def _combine_body(tpm_r, tix_r, tpk_r, tm_r, ts_r,
                  spm_r, six_r, sm_r, ss_r,
                  act_ref, nlp_ref):
    spm_v = spm_r[...]
    six_v = six_r[...]
    sm_v = sm_r[...]
    ss_v = ss_r[...]

    spm = jnp.max(spm_v, axis=1)
    win = spm_v == spm[:, None]
    sidx = jnp.min(jnp.where(win, six_v, jnp.int32(2**30)), axis=1)
    flatw = (lax.broadcasted_iota(jnp.uint32, (B,), 0) * np.uint32(N)
             + sidx.astype(jnp.uint32))
    spick = spm - _gumbel_from_bits(_threefry_bits(flatw))
    sm = jnp.max(sm_v, axis=1)
    ss = jnp.sum(ss_v * jnp.exp(sm_v - sm[:, None]), axis=1)

    tpm = tpm_r[...]
    use_sc = spm > tpm
    idx = jnp.where(use_sc, sidx, tix_r[...])
    pick = jnp.where(use_sc, spick, tpk_r[...])

    tm = tm_r[...]
    mm = jnp.maximum(tm, sm)
    z = ts_r[...] * jnp.exp(tm - mm) + ss * jnp.exp(sm - mm)

    act_ref[...] = idx
    nlp_ref[...] = (mm + jnp.log(z)) - pick


def _combine(tc, sc):
    sc2d = [x.reshape(B, SLOTS) for x in sc]
    return pl.pallas_call(
        _combine_body,
        out_shape=[
            jax.ShapeDtypeStruct((B,), jnp.int32),
            jax.ShapeDtypeStruct((B,), jnp.float32),
        ],
    )(*tc, *sc2d)


@jax.jit
def kernel(logits):
    sc = _sc_partials(logits)
    tc = _tc_partials(logits)
    action, neglogprob = _combine(tc, sc)
    return action, neglogprob


# speedup vs baseline: 1.0159x; 1.0159x over previous
import jax
import jax.numpy as jnp
import numpy as np
from jax import lax
from jax.experimental import pallas as pl
from jax.experimental.pallas import tpu as pltpu
from jax.experimental.pallas import tpu_sc as plsc

B = 32
N = 1000000

BLK = 4096
CH = 128
NCH = BLK // CH

W_SC = 22528
NSLICE = 8
L_SC = W_SC * NSLICE
C0 = 819200
SC_END = C0 + L_SC

NFULL = C0 // BLK
TBLK = SC_END // BLK
NB = NFULL + 1
TAIL = N - SC_END
NCH_TAIL = (TAIL + CH - 1) // CH

SC_NC = 2
SC_NS = 16
NW = SC_NC * SC_NS
LANES = 16
RG = 8
CSC = 2048
NCHUNK = W_SC // CSC
GROUPS = CSC // LANES
NSET = 2
GPAIRS = GROUPS // NSET
SC_OUT = B * NSLICE * NSET * LANES

_TINY = np.float32(np.finfo(np.float32).tiny)
_K0 = np.uint32(0)
_K1 = np.uint32(42)
_K2 = np.uint32(np.uint32(0) ^ np.uint32(42) ^ np.uint32(0x1BD11BDA))
_KS = (_K0, _K1, _K2)
_ROT = ((13, 15, 26, 6), (17, 29, 16, 24))
_INJ = ((1, 2), (2, 0), (0, 1), (1, 2), (2, 0))


def _rotl(x, r):
    return (x << np.uint32(r)) | (x >> np.uint32(32 - r))


def _threefry_bits(cnt):
    x1i = cnt + _KS[1]
    x0 = x1i
    x1 = _rotl(x1i, _ROT[0][0]) ^ x1i
    first = True
    for g in range(5):
        for r in _ROT[g % 2]:
            if first:
                first = False
                continue
            x0 = x0 + x1
            x1 = _rotl(x1, r) ^ x0
        a, b = _INJ[g]
        x0 = x0 + _KS[a]
        x1 = x1 + (_KS[b] + np.uint32(g + 1))
    return x0 ^ x1


def _gumbel_from_bits(bits):
    fb = (bits >> np.uint32(9)) | np.uint32(0x3F800000)
    fl = lax.bitcast_convert_type(fb, jnp.float32) - jnp.float32(1.0)
    u = jnp.maximum(fl, _TINY)
    return -jnp.log(-jnp.log(u))


def _chunk_update(x, col, masked,
                  pmax_ref, pidx_ref, m_ref, s_ref, sl):
    flat = (lax.broadcasted_iota(jnp.uint32, (B, CH), 0) * np.uint32(N)
            + col.astype(jnp.uint32))
    g = _gumbel_from_bits(_threefry_bits(flat))
    p = x + g
    if masked:
        valid = col < N
        p = jnp.where(valid, p, -jnp.inf)
        xs = jnp.where(valid, x, -jnp.inf)
    else:
        xs = x

    pm = pmax_ref[:, sl]
    upd = p > pm
    pmax_ref[:, sl] = jnp.where(upd, p, pm)
    pidx_ref[:, sl] = jnp.where(upd, col, pidx_ref[:, sl])

    mo = m_ref[:, sl]
    nm = jnp.maximum(mo, xs)
    s_ref[:, sl] = s_ref[:, sl] * jnp.exp(mo - nm) + jnp.exp(xs - nm)
    m_ref[:, sl] = nm


def _tc_body(x_ref, pmax_o, pidx_o, pick_o, m_o, s_o,
             pmax_ref, pidx_ref, m_ref, s_ref):
    c = pl.program_id(0)

    @pl.when(c == 0)
    def _init():
        pmax_ref[...] = jnp.full((B, BLK), -jnp.inf, jnp.float32)
        pidx_ref[...] = jnp.zeros((B, BLK), jnp.int32)
        m_ref[...] = jnp.full((B, BLK), -jnp.inf, jnp.float32)
        s_ref[...] = jnp.zeros((B, BLK), jnp.float32)

    ch_iota = lax.broadcasted_iota(jnp.int32, (B, CH), 1)

    @pl.when(c < NFULL)
    def _full():
        base = c * BLK
        for k in range(NCH):
            sl = slice(k * CH, (k + 1) * CH)
            col = (base + k * CH) + ch_iota
            _chunk_update(x_ref[:, sl], col, False,
                          pmax_ref, pidx_ref, m_ref, s_ref, sl)

    @pl.when(c == NFULL)
    def _tail():
        base = TBLK * BLK
        for k in range(NCH_TAIL):
            sl = slice(k * CH, (k + 1) * CH)
            col = (base + k * CH) + ch_iota
            _chunk_update(x_ref[:, sl], col, True,
                          pmax_ref, pidx_ref, m_ref, s_ref, sl)

        pmax_v = pmax_ref[...]
        pidx_v = pidx_ref[...]
        m_v = m_ref[...]
        s_v = s_ref[...]

        bpm = jnp.max(pmax_v, axis=1)
        winners = pmax_v == bpm[:, None]
        idx = jnp.min(jnp.where(winners, pidx_v, jnp.int32(2**30)), axis=1)

        flatw = (lax.broadcasted_iota(jnp.uint32, (B,), 0) * np.uint32(N)
                 + idx.astype(jnp.uint32))
        pick = bpm - _gumbel_from_bits(_threefry_bits(flatw))

        mf = jnp.max(m_v, axis=1)
        z = jnp.sum(s_v * jnp.exp(m_v - mf[:, None]), axis=1)

        pmax_o[...] = bpm
        pidx_o[...] = idx
        pick_o[...] = pick
        m_o[...] = mf
        s_o[...] = z


def _tc_partials(logits):
    return pl.pallas_call(
        _tc_body,
        grid=(NB,),
        in_specs=[pl.BlockSpec(
            (B, BLK), lambda c: (0, jnp.where(c == NFULL, TBLK, c)))],
        out_specs=[pl.BlockSpec((B,), lambda c: (0,))] * 5,
        out_shape=[
            jax.ShapeDtypeStruct((B,), jnp.float32),
            jax.ShapeDtypeStruct((B,), jnp.int32),
            jax.ShapeDtypeStruct((B,), jnp.float32),
            jax.ShapeDtypeStruct((B,), jnp.float32),
            jax.ShapeDtypeStruct((B,), jnp.float32),
        ],
        scratch_shapes=[
            pltpu.VMEM((B, BLK), jnp.float32),
            pltpu.VMEM((B, BLK), jnp.int32),
            pltpu.VMEM((B, BLK), jnp.float32),
            pltpu.VMEM((B, BLK), jnp.float32),
        ],
    )(logits)


_LOG_P = (0.09152395278215408, -0.13937802612781525, 0.14647668600082397,
          -0.16609898209571838, 0.19987723231315613, -0.25000593066215515,
          0.3333338797092438)
_SQRTHF = np.float32(0.70710678118654752440)


def _sc_logf(x):
    bits = lax.bitcast_convert_type(x, jnp.uint32)
    e = (bits >> np.uint32(23)).astype(jnp.int32) - 127
    mb = (bits & np.uint32(0x7FFFFF)) | np.uint32(0x3F800000)
    m = lax.bitcast_convert_type(mb, jnp.float32)
    m01 = m * jnp.float32(0.5)
    small = m01 < _SQRTHF
    e2 = jnp.where(small, e, e + 1)
    f = jnp.where(small, m - jnp.float32(1.0), m01 - jnp.float32(1.0))
    z = f * f
    y = jnp.full_like(f, _LOG_P[0])
    for cc in _LOG_P[1:]:
        y = y * f + jnp.float32(cc)
    y = y * f * z
    ef = e2.astype(jnp.float32)
    y = y + ef * jnp.float32(-2.12194440e-4)
    y = y - jnp.float32(0.5) * z
    return (f + y) + ef * jnp.float32(0.693359375)


def _sc_gumbel_from_bits(bits):
    fb = (bits >> np.uint32(9)) | np.uint32(0x3F800000)
    fl = lax.bitcast_convert_type(fb, jnp.float32) - jnp.float32(1.0)
    u = jnp.maximum(fl, _TINY)
    return -_sc_logf(-_sc_logf(u))


def _sc_body(logits_hbm, pmax_hbm, pidx_hbm, m_hbm, s_hbm,
             buf, apm, aix, am, asum, stage_f, stage_i):
    wid = lax.axis_index("s") * SC_NC + lax.axis_index("c")
    a = wid // NSLICE
    bslc = wid % NSLICE
    row0 = a * RG
    colw0 = C0 + bslc * W_SC
    lane = lax.iota(jnp.int32, LANES)

    for r in range(RG):
        for k in range(NSET):
            apm[r, k] = jnp.full((LANES,), -jnp.inf, jnp.float32)
            aix[r, k] = jnp.zeros((LANES,), jnp.int32)
            am[r, k] = jnp.full((LANES,), -jnp.inf, jnp.float32)
            asum[r, k] = jnp.zeros((LANES,), jnp.float32)

    def chunk(i, _):
        colbase = colw0 + i * CSC
        pltpu.sync_copy(
            logits_hbm.at[pl.ds(row0, RG), pl.ds(colbase, CSC)], buf)

        def per_row(r, _):
            rowN = (row0 + r) * N

            def grp(j, carry):
                out = []
                for k in range(NSET):
                    pm, pix, m, s = carry[4 * k:4 * k + 4]
                    jj = j * NSET + k
                    x = buf[r, pl.ds(jj * LANES, LANES)]
                    col = (colbase + jj * LANES) + lane
                    flat = (rowN + col).astype(jnp.uint32)
                    g = _sc_gumbel_from_bits(_threefry_bits(flat))
                    p = x + g
                    upd = p > pm
                    pm = jnp.where(upd, p, pm)
                    pix = jnp.where(upd, col, pix)
                    nm = jnp.maximum(m, x)
                    s = s * jnp.exp(m - nm) + jnp.exp(x - nm)
                    out += [pm, pix, nm, s]
                return tuple(out)

            carry = tuple(ref[r, k]
                          for k in range(NSET)
                          for ref in (apm, aix, am, asum))
            res = lax.fori_loop(0, GPAIRS, grp, carry, unroll=4)
            for k in range(NSET):
                apm[r, k], aix[r, k], am[r, k], asum[r, k] = (
                    res[4 * k:4 * k + 4])
            return 0

        return lax.fori_loop(0, RG, per_row, 0)

    lax.fori_loop(0, NCHUNK, chunk, 0)

    for r in range(RG):
        for k in range(NSET):
            off = (((row0 + r) * NSLICE + bslc) * NSET + k) * LANES
            stage_f[...] = apm[r, k]
            pltpu.sync_copy(stage_f, pmax_hbm.at[pl.ds(off, LANES)])
            stage_i[...] = aix[r, k]
            pltpu.sync_copy(stage_i, pidx_hbm.at[pl.ds(off, LANES)])
            stage_f[...] = am[r, k]
            pltpu.sync_copy(stage_f, m_hbm.at[pl.ds(off, LANES)])
            stage_f[...] = asum[r, k]
            pltpu.sync_copy(stage_f, s_hbm.at[pl.ds(off, LANES)])


def _sc_partials(logits):
    mesh = plsc.VectorSubcoreMesh(core_axis_name="c", subcore_axis_name="s")
    f32 = jnp.float32
    run = pl.kernel(
        _sc_body,
        out_type=[
            jax.ShapeDtypeStruct((SC_OUT,), f32),
            jax.ShapeDtypeStruct((SC_OUT,), jnp.int32),
            jax.ShapeDtypeStruct((SC_OUT,), f32),
            jax.ShapeDtypeStruct((SC_OUT,), f32),
        ],
        mesh=mesh,
        scratch_types=[
            pltpu.VMEM((RG, CSC), f32),
            pltpu.VMEM((RG, NSET, LANES), f32),
            pltpu.VMEM((RG, NSET, LANES), jnp.int32),
            pltpu.VMEM((RG, NSET, LANES), f32),
            pltpu.VMEM((RG, NSET, LANES), f32),
            pltpu.VMEM((LANES,), f32),
            pltpu.VMEM((LANES,), jnp.int32),
        ],
    )
    return run(logits)


SLOTS = NSLICE * NSET * LANES


def _combine_body(tpm_r, tix_r, tpk_r, tm_r, ts_r,
                  spm_r, six_r, sm_r, ss_r,
                  act_ref, nlp_ref):
    spm_v = spm_r[...]
    six_v = six_r[...]
    sm_v = sm_r[...]
    ss_v = ss_r[...]

    spm = jnp.max(spm_v, axis=1)
    win = spm_v == spm[:, None]
    sidx = jnp.min(jnp.where(win, six_v, jnp.int32(2**30)), axis=1)
    flatw = (lax.broadcasted_iota(jnp.uint32, (B,), 0) * np.uint32(N)
             + sidx.astype(jnp.uint32))
    spick = spm - _gumbel_from_bits(_threefry_bits(flatw))
    sm = jnp.max(sm_v, axis=1)
    ss = jnp.sum(ss_v * jnp.exp(sm_v - sm[:, None]), axis=1)

    tpm = tpm_r[...]
    use_sc = spm > tpm
    idx = jnp.where(use_sc, sidx, tix_r[...])
    pick = jnp.where(use_sc, spick, tpk_r[...])

    tm = tm_r[...]
    mm = jnp.maximum(tm, sm)
    z = ts_r[...] * jnp.exp(tm - mm) + ss * jnp.exp(sm - mm)

    act_ref[...] = idx
    nlp_ref[...] = (mm + jnp.log(z)) - pick


def _combine(tc, sc):
    sc2d = [x.reshape(B, SLOTS) for x in sc]
    return pl.pallas_call(
        _combine_body,
        out_shape=[
            jax.ShapeDtypeStruct((B,), jnp.int32),
            jax.ShapeDtypeStruct((B,), jnp.float32),
        ],
    )(*tc, *sc2d)


@jax.jit
def kernel(logits):
    sc = _sc_partials(logits)
    tc = _tc_partials(logits)
    action, neglogprob = _combine(tc, sc)
    return action, neglogprob
